# baseline (speedup 1.0000x reference)
import jax
import jax.numpy as jnp
from jax import lax
from jax.experimental import pallas as pl
from jax.experimental.pallas import tpu as pltpu
from jax.experimental.pallas import tpu_sc as plsc

_NC = 2
_NS = 16
_NW = _NC * _NS

_BW = 128
_L = 16


def _transpose_block(buf, tbuf, row_vecs):
    def dt_body(dt, carry):
        for dr in range(8):
            col = jnp.full((_L,), dt * 8 + dr, jnp.int32)
            for blk in range(8):
                v = plsc.load_gather(buf, [row_vecs[blk], col])
                tbuf[dt, dr, pl.ds(blk * _L, _L)] = v
        return carry

    lax.fori_loop(0, 8, dt_body, 0)


def _gather_body(idx_hbm, table_hbm, out_hbm, idx_v, buf_a, buf_b,
                 tb_a, tb_b, ga_sem, gb_sem, oa_sem, ob_sem):
    seq = idx_hbm.shape[0]
    n_pairs = seq // 2

    wid = lax.axis_index("s") * _NC + lax.axis_index("c")
    col0 = pl.multiple_of(wid * _BW, _BW)

    pltpu.sync_copy(idx_hbm.at[:, pl.ds(col0, _BW)], idx_v)

    row_vecs = [lax.iota(jnp.int32, _L) + (blk * _L) for blk in range(8)]

    def fire_gather(s, buf, gsem):
        return pltpu.async_copy(table_hbm.at[idx_v.at[s]], buf, gsem)

    def wait_gather(buf, gsem):
        pltpu.make_async_copy(table_hbm.at[idx_v.at[0]], buf, gsem).wait()

    def out_dst(s):
        return out_hbm.at[s, :, wid]

    def fire_out(s, tbuf, osem):
        pltpu.async_copy(tbuf, out_dst(s), osem)

    def wait_out(s, tbuf, osem):
        pltpu.make_async_copy(tbuf, out_dst(s), osem).wait()

    fire_gather(0, buf_a, ga_sem)

    def pair_body(i, carry):
        sa = 2 * i
        sb = 2 * i + 1

        fire_gather(sb, buf_b, gb_sem)
        wait_gather(buf_a, ga_sem)

        @pl.when(i > 0)
        def _():
            wait_out(sa, tb_a, oa_sem)

        _transpose_block(buf_a, tb_a, row_vecs)
        fire_out(sa, tb_a, oa_sem)

        @pl.when(i + 1 < n_pairs)
        def _():
            fire_gather(sb + 1, buf_a, ga_sem)

        wait_gather(buf_b, gb_sem)

        @pl.when(i > 0)
        def _():
            wait_out(sb, tb_b, ob_sem)

        _transpose_block(buf_b, tb_b, row_vecs)
        fire_out(sb, tb_b, ob_sem)
        return carry

    lax.fori_loop(0, n_pairs, pair_body, 0)

    wait_out(seq - 2, tb_a, oa_sem)
    wait_out(seq - 1, tb_b, ob_sem)


@jax.jit
def _gather(idx_t, table_lin):
    seq, batch = idx_t.shape
    d = table_lin.shape[1]
    mesh = plsc.VectorSubcoreMesh(core_axis_name="c", subcore_axis_name="s")
    x = pl.kernel(
        _gather_body,
        out_type=jax.ShapeDtypeStruct((seq, d // 8, batch // _BW, 8, _BW),
                                      jnp.float32),
        mesh=mesh,
        scratch_types=[
            pltpu.VMEM((seq, _BW), jnp.int32),
            pltpu.VMEM((_BW, d), jnp.float32),
            pltpu.VMEM((_BW, d), jnp.float32),
            pltpu.VMEM((8, 8, _BW), jnp.float32),
            pltpu.VMEM((8, 8, _BW), jnp.float32),
            pltpu.SemaphoreType.DMA,
            pltpu.SemaphoreType.DMA,
            pltpu.SemaphoreType.DMA,
            pltpu.SemaphoreType.DMA,
        ],
        compiler_params=pltpu.CompilerParams(use_tc_tiling_on_sc=False,
                                             needs_layout_passes=False),
    )(idx_t, table_lin)
    return x.transpose(2, 4, 0, 1, 3).reshape(batch, seq, d)


def kernel(indices, table):
    vocab, d = table.shape
    t2 = table.T.reshape(d, vocab // 2, 2).transpose(1, 2, 0)
    t2 = t2.reshape(vocab // 2, 2 * d)
    t2 = lax.optimization_barrier(t2)
    table_lin = t2.reshape(vocab, d)
    return _gather(indices.T, table_lin)
